# baseline (speedup 1.0000x reference)
import functools

import jax
import jax.numpy as jnp
from jax import lax
from jax.experimental import pallas as pl
from jax.experimental.pallas import tpu as pltpu
from jax.experimental.pallas import tpu_sc as plsc

_K = 8192
_D = 256
_N = 32768
_BN = 256
_BK = 1024
_NK = _K // _BK
_NI = _N // _BN

_NW = 32
_BPW = _N // _NW
_CH = 128
_NCH = _BPW // _CH


def _dist_argmin_body(x_ref, w_ref, idx_ref, x2_s, w2_s, rmin_s, ridx_s):
    k = pl.program_id(0)
    i = pl.program_id(1)
    wk = w_ref[...]
    xb = x_ref[...]
    rows = pl.ds(i * _BN, _BN)

    @pl.when(i == 0)
    def _():
        ones = jnp.ones((1, _D), jnp.float32)
        w2_s[pl.ds(k, 1), :] = lax.dot_general(
            ones, wk * wk, (((1,), (1,)), ((), ())),
            preferred_element_type=jnp.float32,
            precision=lax.Precision.HIGHEST)

    @pl.when(k == 0)
    def _():
        x2_s[rows, :] = jnp.sum(xb * xb, axis=1, keepdims=True)

    sim = lax.dot_general(
        xb, wk, (((1,), (1,)), ((), ())),
        preferred_element_type=jnp.float32,
        precision=lax.Precision.DEFAULT)
    d = (x2_s[rows, :] + w2_s[pl.ds(k, 1), :]) - 2.0 * sim
    mloc = jnp.min(d, axis=1, keepdims=True)
    ii = lax.broadcasted_iota(jnp.int32, (_BN, _BK), 1)
    iloc = jnp.min(jnp.where(d <= mloc, ii, _BK), axis=1,
                   keepdims=True) + k * _BK

    @pl.when(k == 0)
    def _():
        rmin_s[rows, :] = mloc
        ridx_s[rows, :] = iloc

    @pl.when(k > 0)
    def _():
        old = rmin_s[rows, :]
        upd = mloc < old
        rmin_s[rows, :] = jnp.where(upd, mloc, old)
        ridx_s[rows, :] = jnp.where(upd, iloc, ridx_s[rows, :])

    @pl.when(k == _NK - 1)
    def _():
        idx_ref[...] = ridx_s[rows, :]


def _tc_argmin(xf, W):
    return pl.pallas_call(
        _dist_argmin_body,
        grid=(_NK, _NI),
        in_specs=[
            pl.BlockSpec((_BN, _D), lambda k, i: (i, 0)),
            pl.BlockSpec((_BK, _D), lambda k, i: (k, 0)),
        ],
        out_specs=pl.BlockSpec((_BN, 1), lambda k, i: (i, 0)),
        out_shape=jax.ShapeDtypeStruct((_N, 1), jnp.int32),
        scratch_shapes=[
            pltpu.VMEM((_N, 1), jnp.float32),
            pltpu.VMEM((_NK, _BK), jnp.float32),
            pltpu.VMEM((_N, 1), jnp.float32),
            pltpu.VMEM((_N, 1), jnp.int32),
        ],
    )(xf, W)


@functools.cache
def _sc_gather_fn():
    @functools.partial(
        pl.kernel,
        mesh=plsc.VectorSubcoreMesh(core_axis_name="c", subcore_axis_name="s"),
        out_type=jax.ShapeDtypeStruct((_N, _D), jnp.float32),
        scratch_types=[
            pltpu.VMEM((_NCH, _CH), jnp.int32),
            pltpu.VMEM((_CH, _D), jnp.float32),
            pltpu.SemaphoreType.DMA,
        ],
    )
    def _sc_gather(w_hbm, idx_hbm, out_hbm, idx_v, rows_v, sem):
        wid = lax.axis_index("s") * 2 + lax.axis_index("c")
        pltpu.sync_copy(idx_hbm.at[pl.ds(wid * _NCH, _NCH)], idx_v)
        for c in range(_NCH):
            pltpu.async_copy(w_hbm.at[idx_v.at[c]], rows_v, sem).wait()
            pltpu.sync_copy(rows_v, out_hbm.at[pl.ds(wid * _BPW + c * _CH, _CH)])

    return _sc_gather


def kernel(x, W):
    xf = x.reshape(-1, _D)
    idx = _tc_argmin(xf, W)
    q = _sc_gather_fn()(W, idx.reshape(_NW * _NCH, _CH))
    return q.reshape(x.shape), idx


# speedup vs baseline: 1.9670x; 1.9670x over previous
import functools

import jax
import jax.numpy as jnp
from jax import lax
from jax.experimental import pallas as pl
from jax.experimental.pallas import tpu as pltpu
from jax.experimental.pallas import tpu_sc as plsc

_K = 8192
_D = 256
_N = 32768
_BN = 256
_NI = _N // _BN

_NW = 32
_BPW = _N // _NW
_CH = 128
_NCH = _BPW // _CH


def _dist_argmin_body(x_ref, w_ref, idx_ref, w2_s):
    w = w_ref[...]

    @pl.when(pl.program_id(0) == 0)
    def _():
        ones = jnp.ones((1, _D), jnp.float32)
        w2t = lax.dot_general(
            ones, w * w, (((1,), (1,)), ((), ())),
            preferred_element_type=jnp.float32,
            precision=lax.Precision.HIGHEST)
        w2_s[...] = jnp.transpose(w2t)

    xb = x_ref[...]
    simT = lax.dot_general(
        w, xb, (((1,), (1,)), ((), ())),
        preferred_element_type=jnp.float32,
        precision=lax.Precision.DEFAULT)
    x2 = jnp.sum(xb * xb, axis=1, keepdims=True)
    x2t = jnp.transpose(x2)
    d = (x2t + w2_s[...]) - 2.0 * simT
    m = jnp.min(d, axis=0, keepdims=True)
    ii = lax.broadcasted_iota(jnp.int32, (_K, _BN), 0)
    idx = jnp.min(jnp.where(d <= m, ii, _K), axis=0, keepdims=True)
    idx_ref[...] = idx[None]


def _tc_argmin(xf, W):
    out = pl.pallas_call(
        _dist_argmin_body,
        grid=(_NI,),
        in_specs=[
            pl.BlockSpec((_BN, _D), lambda i: (i, 0)),
            pl.BlockSpec((_K, _D), lambda i: (0, 0)),
        ],
        out_specs=pl.BlockSpec((1, 1, _BN), lambda i: (i, 0, 0)),
        out_shape=jax.ShapeDtypeStruct((_NI, 1, _BN), jnp.int32),
        scratch_shapes=[pltpu.VMEM((_K, 1), jnp.float32)],
    )(xf, W)
    return out.reshape(_N, 1)


@functools.cache
def _sc_gather_fn():
    @functools.partial(
        pl.kernel,
        mesh=plsc.VectorSubcoreMesh(core_axis_name="c", subcore_axis_name="s"),
        out_type=jax.ShapeDtypeStruct((_N, _D), jnp.float32),
        scratch_types=[
            pltpu.VMEM((_NCH, _CH), jnp.int32),
            pltpu.VMEM((_CH, _D), jnp.float32),
            pltpu.SemaphoreType.DMA,
        ],
    )
    def _sc_gather(w_hbm, idx_hbm, out_hbm, idx_v, rows_v, sem):
        wid = lax.axis_index("s") * 2 + lax.axis_index("c")
        pltpu.sync_copy(idx_hbm.at[pl.ds(wid * _NCH, _NCH)], idx_v)
        for c in range(_NCH):
            pltpu.async_copy(w_hbm.at[idx_v.at[c]], rows_v, sem).wait()
            pltpu.sync_copy(rows_v, out_hbm.at[pl.ds(wid * _BPW + c * _CH, _CH)])

    return _sc_gather


def kernel(x, W):
    xf = x.reshape(-1, _D)
    idx = _tc_argmin(xf, W)
    q = _sc_gather_fn()(W, idx.reshape(_NW * _NCH, _CH))
    return q.reshape(x.shape), idx


# speedup vs baseline: 2.8084x; 1.4277x over previous
import functools

import jax
import jax.numpy as jnp
from jax import lax
from jax.experimental import pallas as pl
from jax.experimental.pallas import tpu as pltpu
from jax.experimental.pallas import tpu_sc as plsc

_K = 8192
_D = 256
_N = 32768
_BN = 512
_NI = _N // _BN

_NW = 32
_BPW = _N // _NW
_CH = 128
_NCH = _BPW // _CH


def _dist_argmin_body(x_ref, w_ref, idx_ref, w2_ref):
    @pl.when(pl.program_id(0) == 0)
    def _():
        w = w_ref[...]
        ones = jnp.ones((1, _D), jnp.float32)
        w2_ref[...] = lax.dot_general(
            ones, w * w, (((1,), (1,)), ((), ())),
            preferred_element_type=jnp.float32,
            precision=lax.Precision.HIGHEST)

    xb = x_ref[...]
    sim = lax.dot_general(
        xb, w_ref[...], (((1,), (1,)), ((), ())),
        preferred_element_type=jnp.float32,
        precision=lax.Precision.DEFAULT)
    x2 = jnp.sum(xb * xb, axis=1, keepdims=True)
    d = (x2 + w2_ref[...]) - 2.0 * sim
    idx_ref[...] = jnp.argmin(d, axis=1).astype(jnp.int32)[:, None]


def _tc_argmin(xf, W):
    return pl.pallas_call(
        _dist_argmin_body,
        grid=(_NI,),
        in_specs=[
            pl.BlockSpec((_BN, _D), lambda i: (i, 0)),
            pl.BlockSpec((_K, _D), lambda i: (0, 0)),
        ],
        out_specs=pl.BlockSpec((_BN, 1), lambda i: (i, 0)),
        out_shape=jax.ShapeDtypeStruct((_N, 1), jnp.int32),
        scratch_shapes=[pltpu.VMEM((1, _K), jnp.float32)],
    )(xf, W)


@functools.cache
def _sc_gather_fn():
    @functools.partial(
        pl.kernel,
        mesh=plsc.VectorSubcoreMesh(core_axis_name="c", subcore_axis_name="s"),
        out_type=jax.ShapeDtypeStruct((_N, _D), jnp.float32),
        scratch_types=[
            pltpu.VMEM((_NCH, _CH), jnp.int32),
            pltpu.VMEM((_CH, _D), jnp.float32),
            pltpu.SemaphoreType.DMA,
        ],
    )
    def _sc_gather(w_hbm, idx_hbm, out_hbm, idx_v, rows_v, sem):
        wid = lax.axis_index("s") * 2 + lax.axis_index("c")
        pltpu.sync_copy(idx_hbm.at[pl.ds(wid * _NCH, _NCH)], idx_v)
        for c in range(_NCH):
            pltpu.async_copy(w_hbm.at[idx_v.at[c]], rows_v, sem).wait()
            pltpu.sync_copy(rows_v, out_hbm.at[pl.ds(wid * _BPW + c * _CH, _CH)])

    return _sc_gather


def kernel(x, W):
    xf = x.reshape(-1, _D)
    idx = _tc_argmin(xf, W)
    q = _sc_gather_fn()(W, idx.reshape(_NW * _NCH, _CH))
    return q.reshape(x.shape), idx
